# baseline (speedup 1.0000x reference)
import jax
import jax.numpy as jnp
from jax import lax
from jax.experimental import pallas as pl
from jax.experimental.pallas import tpu as pltpu
from jax.experimental.pallas import tpu_sc as plsc

BATCH = 16384
N_FIELDS = 26
VOCAB = 1_000_000
NUM_IDS = BATCH * N_FIELDS
NUM_WORKERS = 32
ROWS_PER_W = BATCH // NUM_WORKERS
IDS_PER_W = ROWS_PER_W * N_FIELDS
LANES = 16
CHUNKS = ROWS_PER_W // LANES


def _flat_body(w_ref, ow_ref, sem):
    copy = pltpu.make_async_copy(w_ref.at[0], ow_ref, sem)
    copy.start()
    copy.wait()


def _flatten_w(wt):
    return pl.pallas_call(
        _flat_body,
        in_specs=[pl.BlockSpec(memory_space=pltpu.MemorySpace.HBM)],
        out_specs=pl.BlockSpec(memory_space=pltpu.MemorySpace.HBM),
        out_shape=jax.ShapeDtypeStruct((VOCAB,), jnp.float32),
        scratch_shapes=[pltpu.SemaphoreType.DMA],
    )(wt)


def _sc_body(idx_hbm, w_hbm, bias_hbm, out_hbm, idx_v, g_v, bias_v, acc_v, sem):
    wid = lax.axis_index("s") * 2 + lax.axis_index("c")
    base = wid * IDS_PER_W

    pltpu.sync_copy(idx_hbm.at[pl.ds(base, IDS_PER_W)], idx_v)
    pltpu.sync_copy(bias_hbm, bias_v)

    pltpu.async_copy(w_hbm.at[idx_v], g_v, sem).wait()

    zeros = jnp.zeros((LANES,), jnp.int32)
    bvec = plsc.load_gather(bias_v, [zeros])
    lane_iota = lax.iota(jnp.int32, LANES) * N_FIELDS

    def chunk_body(c, _):
        off = c * (LANES * N_FIELDS)
        acc = bvec
        for f in range(N_FIELDS):
            acc = acc + plsc.load_gather(g_v, [lane_iota + (off + f)])
        acc_v[pl.ds(c * LANES, LANES)] = acc
        return 0

    lax.fori_loop(0, CHUNKS, chunk_body, 0)

    pltpu.sync_copy(acc_v, out_hbm.at[pl.ds(wid * ROWS_PER_W, ROWS_PER_W)])


@jax.jit
def _sc_call(idx_flat, w_flat, bias):
    mesh = plsc.VectorSubcoreMesh(core_axis_name="c", subcore_axis_name="s")
    fn = pl.kernel(
        _sc_body,
        out_type=jax.ShapeDtypeStruct((BATCH,), jnp.float32),
        mesh=mesh,
        compiler_params=pltpu.CompilerParams(needs_layout_passes=False),
        scratch_types=[
            pltpu.VMEM((IDS_PER_W,), jnp.int32),
            pltpu.VMEM((IDS_PER_W,), jnp.float32),
            pltpu.VMEM((1,), jnp.float32),
            pltpu.VMEM((ROWS_PER_W,), jnp.float32),
            pltpu.SemaphoreType.DMA,
        ],
    )
    return fn(idx_flat, w_flat, bias)


def kernel(indices, weights, bias):
    w_flat = _flatten_w(lax.transpose(weights, (1, 0)))
    out = _sc_call(indices.reshape(-1), w_flat, bias)
    return out.reshape(BATCH, 1)


# speedup vs baseline: 3.0850x; 3.0850x over previous
import jax
import jax.numpy as jnp
from jax import lax
from jax.experimental import pallas as pl
from jax.experimental.pallas import tpu as pltpu
from jax.experimental.pallas import tpu_sc as plsc

BATCH = 16384
N_FIELDS = 26
VOCAB = 1_000_000
NUM_IDS = BATCH * N_FIELDS
NUM_WORKERS = 32
ROWS_PER_W = BATCH // NUM_WORKERS
IDS_PER_W = ROWS_PER_W * N_FIELDS
LANES = 16
CHUNKS = ROWS_PER_W // LANES


def _flat_body(w_ref, ow_ref):
    ow_ref[...] = w_ref[0, :]


def _flatten_w(wt):
    return pl.pallas_call(
        _flat_body,
        in_specs=[pl.BlockSpec((1, VOCAB), lambda: (0, 0))],
        out_specs=pl.BlockSpec((VOCAB,), lambda: (0,)),
        out_shape=jax.ShapeDtypeStruct((VOCAB,), jnp.float32),
    )(wt)


def _sc_body(idx_hbm, w_hbm, bias_hbm, out_hbm, idx_v, g_v, bias_v, acc_v, sem):
    wid = lax.axis_index("s") * 2 + lax.axis_index("c")
    base = wid * IDS_PER_W

    pltpu.sync_copy(idx_hbm.at[pl.ds(base, IDS_PER_W)], idx_v)
    pltpu.sync_copy(bias_hbm, bias_v)

    pltpu.async_copy(w_hbm.at[idx_v], g_v, sem).wait()

    zeros = jnp.zeros((LANES,), jnp.int32)
    bvec = plsc.load_gather(bias_v, [zeros])
    lane_iota = lax.iota(jnp.int32, LANES) * N_FIELDS

    def chunk_body(c, _):
        off = c * (LANES * N_FIELDS)
        acc = bvec
        for f in range(N_FIELDS):
            acc = acc + plsc.load_gather(g_v, [lane_iota + (off + f)])
        acc_v[pl.ds(c * LANES, LANES)] = acc
        return 0

    lax.fori_loop(0, CHUNKS, chunk_body, 0)

    pltpu.sync_copy(acc_v, out_hbm.at[pl.ds(wid * ROWS_PER_W, ROWS_PER_W)])


@jax.jit
def _sc_call(idx_flat, w_flat, bias):
    mesh = plsc.VectorSubcoreMesh(core_axis_name="c", subcore_axis_name="s")
    fn = pl.kernel(
        _sc_body,
        out_type=jax.ShapeDtypeStruct((BATCH,), jnp.float32),
        mesh=mesh,
        compiler_params=pltpu.CompilerParams(needs_layout_passes=False),
        scratch_types=[
            pltpu.VMEM((IDS_PER_W,), jnp.int32),
            pltpu.VMEM((IDS_PER_W,), jnp.float32),
            pltpu.VMEM((1,), jnp.float32),
            pltpu.VMEM((ROWS_PER_W,), jnp.float32),
            pltpu.SemaphoreType.DMA,
        ],
    )
    return fn(idx_flat, w_flat, bias)


def kernel(indices, weights, bias):
    w_flat = _flatten_w(lax.transpose(weights, (1, 0)))
    out = _sc_call(indices.reshape(-1), w_flat, bias)
    return out.reshape(BATCH, 1)


# speedup vs baseline: 4.0053x; 1.2983x over previous
import jax
import jax.numpy as jnp
from jax import lax
from jax.experimental import pallas as pl
from jax.experimental.pallas import tpu as pltpu
from jax.experimental.pallas import tpu_sc as plsc

BATCH = 16384
N_FIELDS = 26
VOCAB = 1_000_000
NUM_IDS = BATCH * N_FIELDS
NUM_WORKERS = 32
ROWS_PER_W = BATCH // NUM_WORKERS
IDS_PER_W = ROWS_PER_W * N_FIELDS
LANES = 16
CHUNKS = ROWS_PER_W // LANES


def _flat_body(w_ref, ow_ref):
    ow_ref[...] = w_ref[0, :]


def _flatten_w(wt):
    return pl.pallas_call(
        _flat_body,
        in_specs=[pl.BlockSpec((1, VOCAB), lambda: (0, 0))],
        out_specs=pl.BlockSpec((VOCAB,), lambda: (0,)),
        out_shape=jax.ShapeDtypeStruct((VOCAB,), jnp.float32),
    )(wt)


def _sc_body(idx_hbm, w_hbm, bias_hbm, out_hbm, idx_v, g_v, bias_v, acc_v, sem):
    wid = lax.axis_index("s") * 2 + lax.axis_index("c")
    row0 = wid * ROWS_PER_W

    for f in range(N_FIELDS):
        pltpu.async_copy(
            idx_hbm.at[pl.ds(f * BATCH + row0, ROWS_PER_W)],
            idx_v.at[pl.ds(f * ROWS_PER_W, ROWS_PER_W)],
            sem,
        ).start()
    pltpu.sync_copy(bias_hbm, bias_v)
    for f in range(N_FIELDS):
        pltpu.async_copy(
            idx_hbm.at[pl.ds(f * BATCH + row0, ROWS_PER_W)],
            idx_v.at[pl.ds(f * ROWS_PER_W, ROWS_PER_W)],
            sem,
        ).wait()

    pltpu.async_copy(w_hbm.at[idx_v], g_v, sem).wait()

    zeros = jnp.zeros((LANES,), jnp.int32)
    bvec = plsc.load_gather(bias_v, [zeros])

    def chunk_body(c, _):
        b0 = c * LANES
        acc = bvec
        for f in range(N_FIELDS):
            acc = acc + g_v[pl.ds(f * ROWS_PER_W + b0, LANES)]
        acc_v[pl.ds(b0, LANES)] = acc
        return 0

    lax.fori_loop(0, CHUNKS, chunk_body, 0)

    pltpu.sync_copy(acc_v, out_hbm.at[pl.ds(row0, ROWS_PER_W)])


@jax.jit
def _sc_call(idx_flat, w_flat, bias):
    mesh = plsc.VectorSubcoreMesh(core_axis_name="c", subcore_axis_name="s")
    fn = pl.kernel(
        _sc_body,
        out_type=jax.ShapeDtypeStruct((BATCH,), jnp.float32),
        mesh=mesh,
        compiler_params=pltpu.CompilerParams(needs_layout_passes=False),
        scratch_types=[
            pltpu.VMEM((IDS_PER_W,), jnp.int32),
            pltpu.VMEM((IDS_PER_W,), jnp.float32),
            pltpu.VMEM((1,), jnp.float32),
            pltpu.VMEM((ROWS_PER_W,), jnp.float32),
            pltpu.SemaphoreType.DMA,
        ],
    )
    return fn(idx_flat, w_flat, bias)


def kernel(indices, weights, bias):
    w_flat = _flatten_w(lax.transpose(weights, (1, 0)))
    out = _sc_call(indices.T.reshape(-1), w_flat, bias)
    return out.reshape(BATCH, 1)
